# baseline (speedup 1.0000x reference)
import jax
import jax.numpy as jnp
from jax import lax
from jax.experimental import pallas as pl
from jax.experimental.pallas import tpu as pltpu, tpu_sc as plsc

B = 4 * 8192
D = 512
NC = 2
NS = 16
NW = NC * NS
BPW = B // NW
CHUNK = 64
NCHUNK = BPW // CHUNK


def _sc_body(x_hbm, table_hbm, out_hbm, idx_v, rows_v, sem):
    wid = lax.axis_index("s") * NC + lax.axis_index("c")
    base = wid * BPW
    pltpu.sync_copy(x_hbm.at[wid], idx_v)
    for j in range(NCHUNK):
        pltpu.async_copy(table_hbm.at[idx_v.at[j]], rows_v, sem).wait()
        pltpu.sync_copy(rows_v, out_hbm.at[pl.ds(base + j * CHUNK, CHUNK)])


def kernel(x, table):
    xf = x.reshape(NW, NCHUNK, CHUNK).astype(jnp.int32)
    out = pl.kernel(
        _sc_body,
        out_type=jax.ShapeDtypeStruct((B, D), jnp.float32),
        mesh=plsc.VectorSubcoreMesh(core_axis_name="c", subcore_axis_name="s"),
        scratch_types=[
            pltpu.VMEM((NCHUNK, CHUNK), jnp.int32),
            pltpu.VMEM((CHUNK, D), jnp.float32),
            pltpu.SemaphoreType.DMA,
        ],
    )(xf, table)
    return out.reshape(x.shape[0], x.shape[1], D)


# speedup vs baseline: 1.0051x; 1.0051x over previous
import jax
import jax.numpy as jnp
from jax import lax
from jax.experimental import pallas as pl
from jax.experimental.pallas import tpu as pltpu, tpu_sc as plsc

B = 4 * 8192
D = 512
NC = 2
NS = 16
NW = NC * NS
BPW = B // NW
CHUNK = 64
NCHUNK = BPW // CHUNK
NBUF = 3


def _sc_body(x_hbm, table_hbm, out_hbm, idx_v, rows_v, gsem, osem):
    wid = lax.axis_index("s") * NC + lax.axis_index("c")
    base = wid * BPW
    pltpu.sync_copy(x_hbm.at[wid], idx_v)

    gathers = [None] * NCHUNK
    outs = [None] * NCHUNK
    for j in range(min(NBUF, NCHUNK)):
        gathers[j] = pltpu.make_async_copy(
            table_hbm.at[idx_v.at[j]], rows_v.at[j % NBUF], gsem)
        gathers[j].start()
    for j in range(NCHUNK):
        b = j % NBUF
        gathers[j].wait()
        outs[j] = pltpu.make_async_copy(
            rows_v.at[b], out_hbm.at[pl.ds(base + j * CHUNK, CHUNK)], osem)
        outs[j].start()
        nj = j + NBUF
        if nj < NCHUNK:
            outs[j].wait()
            gathers[nj] = pltpu.make_async_copy(
                table_hbm.at[idx_v.at[nj]], rows_v.at[b], gsem)
            gathers[nj].start()
    for j in range(max(0, NCHUNK - NBUF), NCHUNK):
        outs[j].wait()


def kernel(x, table):
    xf = x.reshape(NW, NCHUNK, CHUNK).astype(jnp.int32)
    out = pl.kernel(
        _sc_body,
        out_type=jax.ShapeDtypeStruct((B, D), jnp.float32),
        mesh=plsc.VectorSubcoreMesh(core_axis_name="c", subcore_axis_name="s"),
        scratch_types=[
            pltpu.VMEM((NCHUNK, CHUNK), jnp.int32),
            pltpu.VMEM((NBUF, CHUNK, D), jnp.float32),
            pltpu.SemaphoreType.DMA,
            pltpu.SemaphoreType.DMA,
        ],
    )(xf, table)
    return out.reshape(x.shape[0], x.shape[1], D)


# speedup vs baseline: 9.7433x; 9.6938x over previous
import jax
import jax.numpy as jnp
from jax import lax
from jax.experimental import pallas as pl
from jax.experimental.pallas import tpu as pltpu, tpu_sc as plsc

B = 4 * 8192
D = 512
NC = 2
NS = 16
NW = NC * NS
BPW = B // NW
CHUNK = 64
NCHUNK = BPW // CHUNK
NBUF = 3
RPW = 16
REP_ROWS = NW * RPW * 2


def _sc_body(x_hbm, table_hbm, out_hbm, rep_hbm,
             idx_v, tbl_v, rows_v, gsem, osem):
    wid = lax.axis_index("s") * NC + lax.axis_index("c")
    pltpu.sync_copy(x_hbm.at[wid], idx_v)
    pltpu.sync_copy(table_hbm, tbl_v)

    reps = []
    for r in range(RPW):
        c = pltpu.make_async_copy(
            tbl_v, rep_hbm.at[pl.ds((wid * RPW + r) * 2, 2)], osem)
        c.start()
        reps.append(c)

    off = 2 * (wid * RPW) + 2 * lax.iota(jnp.int32, 16)
    for c16 in range(NCHUNK):
        for g in range(CHUNK // 16):
            sl = pl.ds(g * 16, 16)
            idx_v[c16, sl] = idx_v[c16, sl] + off

    for c in reps:
        c.wait()

    base = wid * BPW
    gathers = [None] * NCHUNK
    outs = [None] * NCHUNK
    for j in range(min(NBUF, NCHUNK)):
        gathers[j] = pltpu.make_async_copy(
            rep_hbm.at[idx_v.at[j]], rows_v.at[j % NBUF], gsem)
        gathers[j].start()
    for j in range(NCHUNK):
        b = j % NBUF
        gathers[j].wait()
        outs[j] = pltpu.make_async_copy(
            rows_v.at[b], out_hbm.at[pl.ds(base + j * CHUNK, CHUNK)], osem)
        outs[j].start()
        nj = j + NBUF
        if nj < NCHUNK:
            outs[j].wait()
            gathers[nj] = pltpu.make_async_copy(
                rep_hbm.at[idx_v.at[nj]], rows_v.at[b], gsem)
            gathers[nj].start()
    for j in range(max(0, NCHUNK - NBUF), NCHUNK):
        outs[j].wait()


def kernel(x, table):
    xf = x.reshape(NW, NCHUNK, CHUNK).astype(jnp.int32)
    out, _ = pl.kernel(
        _sc_body,
        out_type=[
            jax.ShapeDtypeStruct((B, D), jnp.float32),
            jax.ShapeDtypeStruct((REP_ROWS, D), jnp.float32),
        ],
        mesh=plsc.VectorSubcoreMesh(core_axis_name="c", subcore_axis_name="s"),
        scratch_types=[
            pltpu.VMEM((NCHUNK, CHUNK), jnp.int32),
            pltpu.VMEM((2, D), jnp.float32),
            pltpu.VMEM((NBUF, CHUNK, D), jnp.float32),
            pltpu.SemaphoreType.DMA,
            pltpu.SemaphoreType.DMA,
        ],
    )(xf, table)
    return out.reshape(x.shape[0], x.shape[1], D)


# speedup vs baseline: 10.1293x; 1.0396x over previous
import jax
import jax.numpy as jnp
from jax import lax
from jax.experimental import pallas as pl
from jax.experimental.pallas import tpu as pltpu, tpu_sc as plsc

B = 4 * 8192
D = 512
NC = 2
NS = 16
NW = NC * NS
BPW = B // NW
CHUNK = 64
NCHUNK = BPW // CHUNK
NBUF = 3
RPW = 32
REP_ROWS = NW * RPW * 2


def _sc_body(x_hbm, table_hbm, out_hbm, rep_hbm,
             idx_v, tbl_v, rows_v, gsem, osem):
    wid = lax.axis_index("s") * NC + lax.axis_index("c")
    pltpu.sync_copy(x_hbm.at[wid], idx_v)
    pltpu.sync_copy(table_hbm, tbl_v)

    reps = []
    for r in range(RPW):
        c = pltpu.make_async_copy(
            tbl_v, rep_hbm.at[pl.ds((wid * RPW + r) * 2, 2)], osem)
        c.start()
        reps.append(c)

    off0 = 2 * (wid * RPW) + 2 * lax.iota(jnp.int32, 16)
    for c16 in range(NCHUNK):
        for g in range(CHUNK // 16):
            sl = pl.ds(g * 16, 16)
            idx_v[c16, sl] = idx_v[c16, sl] + (off0 + (g % 2) * 32)

    for c in reps:
        c.wait()

    base = wid * BPW
    gathers = [None] * NCHUNK
    outs = [None] * NCHUNK
    for j in range(min(NBUF, NCHUNK)):
        gathers[j] = pltpu.make_async_copy(
            rep_hbm.at[idx_v.at[j]], rows_v.at[j % NBUF], gsem)
        gathers[j].start()
    for j in range(NCHUNK):
        b = j % NBUF
        gathers[j].wait()
        outs[j] = pltpu.make_async_copy(
            rows_v.at[b], out_hbm.at[pl.ds(base + j * CHUNK, CHUNK)], osem)
        outs[j].start()
        nj = j + NBUF
        if nj < NCHUNK:
            outs[j].wait()
            gathers[nj] = pltpu.make_async_copy(
                rep_hbm.at[idx_v.at[nj]], rows_v.at[b], gsem)
            gathers[nj].start()
    for j in range(max(0, NCHUNK - NBUF), NCHUNK):
        outs[j].wait()


def kernel(x, table):
    xf = x.reshape(NW, NCHUNK, CHUNK).astype(jnp.int32)
    out, _ = pl.kernel(
        _sc_body,
        out_type=[
            jax.ShapeDtypeStruct((B, D), jnp.float32),
            jax.ShapeDtypeStruct((REP_ROWS, D), jnp.float32),
        ],
        mesh=plsc.VectorSubcoreMesh(core_axis_name="c", subcore_axis_name="s"),
        scratch_types=[
            pltpu.VMEM((NCHUNK, CHUNK), jnp.int32),
            pltpu.VMEM((2, D), jnp.float32),
            pltpu.VMEM((NBUF, CHUNK, D), jnp.float32),
            pltpu.SemaphoreType.DMA,
            pltpu.SemaphoreType.DMA,
        ],
    )(xf, table)
    return out.reshape(x.shape[0], x.shape[1], D)
